# baseline (speedup 1.0000x reference)
import functools

import jax
import jax.numpy as jnp
from jax import lax
from jax.experimental import pallas as pl
from jax.experimental.pallas import tpu as pltpu
from jax.experimental.pallas import tpu_sc as plsc

_B = 16384
_D = 64
_NW = 32
_BPW = _B // _NW
_CHUNK = 128
_NCHUNK = _BPW // _CHUNK

_MLP_BLK = 2048
_TR_BLK = 16384
_NROWS = 1000000


def _transpose_body(tu_ref, ti_ref, out_ref):
    tb = jnp.swapaxes(tu_ref[...].astype(jnp.bfloat16), 0, 1)
    ib = jnp.swapaxes(ti_ref[...].astype(jnp.bfloat16), 0, 1)
    out_ref[:, 0:_D] = tb.astype(jnp.float32)
    out_ref[:, _D:2 * _D] = ib.astype(jnp.float32)


_HGRID = 31
_HROWS = _HGRID * _TR_BLK


def _build_combined(tu, ti, half):
    off = half * _HGRID
    return pl.pallas_call(
        _transpose_body,
        grid=(_HGRID,),
        in_specs=[
            pl.BlockSpec((_D, _TR_BLK), lambda i: (0, i + off)),
            pl.BlockSpec((_D, _TR_BLK), lambda i: (0, i + off)),
        ],
        out_specs=pl.BlockSpec((_TR_BLK, 2 * _D), lambda i: (i, 0)),
        out_shape=jax.ShapeDtypeStruct((_HROWS, 2 * _D), jnp.float32),
    )(tu, ti)


def _gather_body(uidx_hbm, iidx_hbm, comb_hbm, gu_hbm, gi_hbm,
                 idx_u, idx_i, rows, sem):
    wid = lax.axis_index("s") * 2 + lax.axis_index("c")
    base = wid * _BPW
    pltpu.sync_copy(uidx_hbm.at[pl.ds(base, _BPW)], idx_u)
    pltpu.sync_copy(iidx_hbm.at[pl.ds(base, _BPW)], idx_i)
    for idx, out in ((idx_u, gu_hbm), (idx_i, gi_hbm)):
        cps = []
        for j in range(_NCHUNK):
            sl = pl.ds(j * _CHUNK, _CHUNK)
            cps.append(pltpu.async_copy(comb_hbm.at[idx.at[sl]], rows.at[sl], sem))
        for c in cps:
            c.wait()
        pltpu.sync_copy(rows, out.at[pl.ds(base, _BPW)])


@functools.cache
def _sc_gather():
    return pl.kernel(
        _gather_body,
        out_type=(
            jax.ShapeDtypeStruct((_B, 2 * _D), jnp.float32),
            jax.ShapeDtypeStruct((_B, 2 * _D), jnp.float32),
        ),
        mesh=plsc.VectorSubcoreMesh(core_axis_name="c", subcore_axis_name="s"),
        scratch_types=[
            pltpu.VMEM((_BPW,), jnp.int32),
            pltpu.VMEM((_BPW,), jnp.int32),
            pltpu.VMEM((_BPW, 2 * _D), jnp.float32),
            pltpu.SemaphoreType.DMA,
        ],
    )


def _mlp_body(gu0_ref, gu1_ref, gi0_ref, gi1_ref, mu_ref, mi_ref,
              w0u, w0i, b0, w1, b1, w2, b2, w3, b3,
              wo, bo, out_ref):
    hp = jnp.float32
    gu = jnp.where(mu_ref[...] != 0, gu0_ref[...], gu1_ref[...])
    gi = jnp.where(mi_ref[...] != 0, gi0_ref[...], gi1_ref[...])
    h = jnp.dot(gu, w0u[...], preferred_element_type=hp)
    h = h + jnp.dot(gi, w0i[...], preferred_element_type=hp)
    h = jnp.maximum(h + b0[...], 0.0)
    h = jnp.maximum(jnp.dot(h, w1[...], preferred_element_type=hp) + b1[...], 0.0)
    h = jnp.maximum(jnp.dot(h, w2[...], preferred_element_type=hp) + b2[...], 0.0)
    h = jnp.maximum(jnp.dot(h, w3[...], preferred_element_type=hp) + b3[...], 0.0)
    logits = jnp.sum(h * wo[...], axis=1) + bo[0, 0]
    out_ref[...] = 5.0 * jax.nn.sigmoid(logits)


def _mlp(gu0, gu1, gi0, gi1, mu, mi, w0u, w0i, b0, W1, b1, W2, b2, W3, b3,
         wo, bo):
    full = lambda shape: pl.BlockSpec(shape, lambda i: (0,) * len(shape))
    grid = _B // _MLP_BLK
    gblk = lambda: pl.BlockSpec((_MLP_BLK, 2 * _D), lambda i: (i, 0))
    return pl.pallas_call(
        _mlp_body,
        grid=(grid,),
        in_specs=[
            gblk(), gblk(), gblk(), gblk(),
            pl.BlockSpec((_MLP_BLK, 1), lambda i: (i, 0)),
            pl.BlockSpec((_MLP_BLK, 1), lambda i: (i, 0)),
            full(w0u.shape), full(w0i.shape), full(b0.shape),
            full(W1.shape), full(b1.shape),
            full(W2.shape), full(b2.shape),
            full(W3.shape), full(b3.shape),
            full(wo.shape), full(bo.shape),
        ],
        out_specs=pl.BlockSpec((_MLP_BLK,), lambda i: (i,)),
        out_shape=jax.ShapeDtypeStruct((_B,), jnp.float32),
    )(gu0, gu1, gi0, gi1, mu, mi, w0u, w0i, b0, W1, b1, W2, b2, W3, b3,
      wo, bo)


@jax.jit
def kernel(user_input, item_input, user_table, item_table,
           W0, b0, W1, b1, W2, b2, W3, b3, Wo, bo):
    tu, ti = user_table.T, item_table.T
    pu0 = jnp.clip(user_input, 0, _HROWS - 1)
    pi0 = jnp.clip(item_input, 0, _HROWS - 1)
    pu1 = jnp.clip(user_input - _HROWS, 0, _HROWS - 1)
    pi1 = jnp.clip(item_input - _HROWS, 0, _HROWS - 1)
    comb0 = _build_combined(tu, ti, 0)
    gu0, gi0 = _sc_gather()(pu0, pi0, comb0)
    comb1 = _build_combined(tu, ti, 1)
    gu1, gi1 = _sc_gather()(pu1, pi1, comb1)
    mu = (user_input < _HROWS).astype(jnp.int32).reshape(-1, 1)
    mi = (item_input < _HROWS).astype(jnp.int32).reshape(-1, 1)
    z = jnp.zeros((_D, W0.shape[1]), W0.dtype)
    w0u = jnp.concatenate([W0[:_D, :], z], axis=0)
    w0i = jnp.concatenate([z, W0[_D:, :]], axis=0)
    return _mlp(
        gu0, gu1, gi0, gi1, mu, mi, w0u, w0i, b0.reshape(1, -1),
        W1, b1.reshape(1, -1),
        W2, b2.reshape(1, -1),
        W3, b3.reshape(1, -1),
        Wo.reshape(1, -1), bo.reshape(1, 1),
    )


# speedup vs baseline: 4.4906x; 4.4906x over previous
import functools

import jax
import jax.numpy as jnp
from jax import lax
from jax.experimental import pallas as pl
from jax.experimental.pallas import tpu as pltpu
from jax.experimental.pallas import tpu_sc as plsc

_B = 16384
_D = 64
_NW = 32
_BPW = _B // _NW
_CHUNK = 128
_NCHUNK = _BPW // _CHUNK

_MLP_BLK = 2048
_TR_BLK = 16384
_NROWS = 1000000


def _transpose_body(tu_ref, ti_ref, out_ref):
    tb = jnp.swapaxes(tu_ref[...].astype(jnp.bfloat16), 0, 1)
    ib = jnp.swapaxes(ti_ref[...].astype(jnp.bfloat16), 0, 1)
    out_ref[:, 0:_D] = tb.astype(jnp.float32)
    out_ref[:, _D:2 * _D] = ib.astype(jnp.float32)


def _build_combined(tu, ti):
    grid = (_NROWS + _TR_BLK - 1) // _TR_BLK
    return pl.pallas_call(
        _transpose_body,
        grid=(grid,),
        in_specs=[
            pl.BlockSpec((_D, _TR_BLK), lambda i: (0, i)),
            pl.BlockSpec((_D, _TR_BLK), lambda i: (0, i)),
        ],
        out_specs=pl.BlockSpec((_TR_BLK, 2 * _D), lambda i: (i, 0)),
        out_shape=jax.ShapeDtypeStruct((_NROWS, 2 * _D), jnp.float32),
    )(tu, ti)


def _gather_body(uidx_hbm, iidx_hbm, comb_hbm, gu_hbm, gi_hbm,
                 idx_u, idx_i, rows, sem):
    wid = lax.axis_index("s") * 2 + lax.axis_index("c")
    base = wid * _BPW
    pltpu.sync_copy(uidx_hbm.at[pl.ds(base, _BPW)], idx_u)
    pltpu.sync_copy(iidx_hbm.at[pl.ds(base, _BPW)], idx_i)
    for idx, out in ((idx_u, gu_hbm), (idx_i, gi_hbm)):
        cps = []
        for j in range(_NCHUNK):
            sl = pl.ds(j * _CHUNK, _CHUNK)
            cps.append(pltpu.async_copy(comb_hbm.at[idx.at[sl]], rows.at[sl], sem))
        for c in cps:
            c.wait()
        pltpu.sync_copy(rows, out.at[pl.ds(base, _BPW)])


@functools.cache
def _sc_gather():
    return pl.kernel(
        _gather_body,
        out_type=(
            jax.ShapeDtypeStruct((_B, 2 * _D), jnp.float32),
            jax.ShapeDtypeStruct((_B, 2 * _D), jnp.float32),
        ),
        mesh=plsc.VectorSubcoreMesh(core_axis_name="c", subcore_axis_name="s"),
        scratch_types=[
            pltpu.VMEM((_BPW,), jnp.int32),
            pltpu.VMEM((_BPW,), jnp.int32),
            pltpu.VMEM((_BPW, 2 * _D), jnp.float32),
            pltpu.SemaphoreType.DMA,
        ],
    )


def _mlp_body(gu_ref, gi_ref, w0u, w0i, b0, w1, b1, w2, b2, w3, b3,
              wo, bo, out_ref):
    hp = jnp.float32
    h = jnp.dot(gu_ref[...], w0u[...], preferred_element_type=hp)
    h = h + jnp.dot(gi_ref[...], w0i[...], preferred_element_type=hp)
    h = jnp.maximum(h + b0[...], 0.0)
    h = jnp.maximum(jnp.dot(h, w1[...], preferred_element_type=hp) + b1[...], 0.0)
    h = jnp.maximum(jnp.dot(h, w2[...], preferred_element_type=hp) + b2[...], 0.0)
    h = jnp.maximum(jnp.dot(h, w3[...], preferred_element_type=hp) + b3[...], 0.0)
    logits = jnp.sum(h * wo[...], axis=1) + bo[0, 0]
    out_ref[...] = 5.0 * jax.nn.sigmoid(logits)


def _mlp(gu, gi, w0u, w0i, b0, W1, b1, W2, b2, W3, b3, wo, bo):
    full = lambda shape: pl.BlockSpec(shape, lambda i: (0,) * len(shape))
    grid = _B // _MLP_BLK
    return pl.pallas_call(
        _mlp_body,
        grid=(grid,),
        in_specs=[
            pl.BlockSpec((_MLP_BLK, 2 * _D), lambda i: (i, 0)),
            pl.BlockSpec((_MLP_BLK, 2 * _D), lambda i: (i, 0)),
            full(w0u.shape), full(w0i.shape), full(b0.shape),
            full(W1.shape), full(b1.shape),
            full(W2.shape), full(b2.shape),
            full(W3.shape), full(b3.shape),
            full(wo.shape), full(bo.shape),
        ],
        out_specs=pl.BlockSpec((_MLP_BLK,), lambda i: (i,)),
        out_shape=jax.ShapeDtypeStruct((_B,), jnp.float32),
    )(gu, gi, w0u, w0i, b0, W1, b1, W2, b2, W3, b3, wo, bo)


@jax.jit
def kernel(user_input, item_input, user_table, item_table,
           W0, b0, W1, b1, W2, b2, W3, b3, Wo, bo):
    comb = _build_combined(user_table.T, item_table.T)
    gu, gi = _sc_gather()(user_input, item_input, comb)
    z = jnp.zeros((_D, W0.shape[1]), W0.dtype)
    w0u = jnp.concatenate([W0[:_D, :], z], axis=0)
    w0i = jnp.concatenate([z, W0[_D:, :]], axis=0)
    return _mlp(
        gu, gi, w0u, w0i, b0.reshape(1, -1),
        W1, b1.reshape(1, -1),
        W2, b2.reshape(1, -1),
        W3, b3.reshape(1, -1),
        Wo.reshape(1, -1), bo.reshape(1, 1),
    )


# speedup vs baseline: 4.5256x; 1.0078x over previous
import functools

import jax
import jax.numpy as jnp
from jax import lax
from jax.experimental import pallas as pl
from jax.experimental.pallas import tpu as pltpu
from jax.experimental.pallas import tpu_sc as plsc

_B = 16384
_D = 64
_NW = 32
_BPW = _B // _NW
_CHUNK = 128
_NCHUNK = _BPW // _CHUNK

_MLP_BLK = 2048
_TR_BLK = 24576
_NROWS = 1000000


def _transpose_body(tu_ref, ti_ref, out_ref):
    tb = jnp.swapaxes(tu_ref[...].astype(jnp.bfloat16), 0, 1)
    ib = jnp.swapaxes(ti_ref[...].astype(jnp.bfloat16), 0, 1)
    out_ref[:, 0:_D] = tb.astype(jnp.float32)
    out_ref[:, _D:2 * _D] = ib.astype(jnp.float32)


def _build_combined(tu, ti):
    grid = (_NROWS + _TR_BLK - 1) // _TR_BLK
    return pl.pallas_call(
        _transpose_body,
        grid=(grid,),
        in_specs=[
            pl.BlockSpec((_D, _TR_BLK), lambda i: (0, i)),
            pl.BlockSpec((_D, _TR_BLK), lambda i: (0, i)),
        ],
        out_specs=pl.BlockSpec((_TR_BLK, 2 * _D), lambda i: (i, 0)),
        out_shape=jax.ShapeDtypeStruct((_NROWS, 2 * _D), jnp.float32),
    )(tu, ti)


def _gather_body(uidx_hbm, iidx_hbm, comb_hbm, gu_hbm, gi_hbm,
                 idx_u, idx_i, rows, sem):
    wid = lax.axis_index("s") * 2 + lax.axis_index("c")
    base = wid * _BPW
    pltpu.sync_copy(uidx_hbm.at[pl.ds(base, _BPW)], idx_u)
    pltpu.sync_copy(iidx_hbm.at[pl.ds(base, _BPW)], idx_i)
    for idx, out in ((idx_u, gu_hbm), (idx_i, gi_hbm)):
        cps = []
        for j in range(_NCHUNK):
            sl = pl.ds(j * _CHUNK, _CHUNK)
            cps.append(pltpu.async_copy(comb_hbm.at[idx.at[sl]], rows.at[sl], sem))
        for c in cps:
            c.wait()
        pltpu.sync_copy(rows, out.at[pl.ds(base, _BPW)])


@functools.cache
def _sc_gather():
    return pl.kernel(
        _gather_body,
        out_type=(
            jax.ShapeDtypeStruct((_B, 2 * _D), jnp.float32),
            jax.ShapeDtypeStruct((_B, 2 * _D), jnp.float32),
        ),
        mesh=plsc.VectorSubcoreMesh(core_axis_name="c", subcore_axis_name="s"),
        scratch_types=[
            pltpu.VMEM((_BPW,), jnp.int32),
            pltpu.VMEM((_BPW,), jnp.int32),
            pltpu.VMEM((_BPW, 2 * _D), jnp.float32),
            pltpu.SemaphoreType.DMA,
        ],
    )


def _mlp_body(gu_ref, gi_ref, w0u, w0i, b0, w1, b1, w2, b2, w3, b3,
              wo, bo, out_ref):
    hp = jnp.float32
    h = jnp.dot(gu_ref[...], w0u[...], preferred_element_type=hp)
    h = h + jnp.dot(gi_ref[...], w0i[...], preferred_element_type=hp)
    h = jnp.maximum(h + b0[...], 0.0)
    h = jnp.maximum(jnp.dot(h, w1[...], preferred_element_type=hp) + b1[...], 0.0)
    h = jnp.maximum(jnp.dot(h, w2[...], preferred_element_type=hp) + b2[...], 0.0)
    h = jnp.maximum(jnp.dot(h, w3[...], preferred_element_type=hp) + b3[...], 0.0)
    logits = jnp.sum(h * wo[...], axis=1) + bo[0, 0]
    out_ref[...] = 5.0 * jax.nn.sigmoid(logits)


def _mlp(gu, gi, w0u, w0i, b0, W1, b1, W2, b2, W3, b3, wo, bo):
    full = lambda shape: pl.BlockSpec(shape, lambda i: (0,) * len(shape))
    grid = _B // _MLP_BLK
    return pl.pallas_call(
        _mlp_body,
        grid=(grid,),
        in_specs=[
            pl.BlockSpec((_MLP_BLK, 2 * _D), lambda i: (i, 0)),
            pl.BlockSpec((_MLP_BLK, 2 * _D), lambda i: (i, 0)),
            full(w0u.shape), full(w0i.shape), full(b0.shape),
            full(W1.shape), full(b1.shape),
            full(W2.shape), full(b2.shape),
            full(W3.shape), full(b3.shape),
            full(wo.shape), full(bo.shape),
        ],
        out_specs=pl.BlockSpec((_MLP_BLK,), lambda i: (i,)),
        out_shape=jax.ShapeDtypeStruct((_B,), jnp.float32),
    )(gu, gi, w0u, w0i, b0, W1, b1, W2, b2, W3, b3, wo, bo)


@jax.jit
def kernel(user_input, item_input, user_table, item_table,
           W0, b0, W1, b1, W2, b2, W3, b3, Wo, bo):
    comb = _build_combined(user_table.T, item_table.T)
    gu, gi = _sc_gather()(user_input, item_input, comb)
    z = jnp.zeros((_D, W0.shape[1]), W0.dtype)
    w0u = jnp.concatenate([W0[:_D, :], z], axis=0)
    w0i = jnp.concatenate([z, W0[_D:, :]], axis=0)
    return _mlp(
        gu, gi, w0u, w0i, b0.reshape(1, -1),
        W1, b1.reshape(1, -1),
        W2, b2.reshape(1, -1),
        W3, b3.reshape(1, -1),
        Wo.reshape(1, -1), bo.reshape(1, 1),
    )
